# baseline (speedup 1.0000x reference)
import jax
import jax.numpy as jnp
from jax import lax
from jax.experimental import pallas as pl
from jax.experimental.pallas import tpu as pltpu

N = 1000000
CHB = 65536
NB = 16
A = CHB // 128
BIG = 2**30
PATCH = 128


def _view2d(v):
    return pltpu.einshape("(ab)->ab", v, a=A, b=128,
                          assert_is_tile_preserving=True)


def _view1d(v):
    return pltpu.einshape("ab->(ab)", v, assert_is_tile_preserving=True)


def _lin(i):
    r = lax.broadcasted_iota(jnp.int32, (A, 128), 0)
    c = lax.broadcasted_iota(jnp.int32, (A, 128), 1)
    return r * 128 + c + i * CHB


def _k1_body(x_ref, out_ref, idx_ref, max_ref, lm_ref):
    i = pl.program_id(0)

    @pl.when(i < NB - 1)
    def _plain_max():
        lm_ref[0] = jnp.max(_view2d(x_ref[...]))

    @pl.when(i == NB - 1)
    def _masked_max():
        xv = _view2d(x_ref[...])
        lm_ref[0] = jnp.max(jnp.where(_lin(i) < N, xv, -jnp.inf))

    lm = lm_ref[0]

    @pl.when((i == 0) | (lm > max_ref[0]))
    def _new_max():
        xv = _view2d(x_ref[...])
        cand = jnp.where(xv == lm, _lin(i), BIG)
        max_ref[0] = lm
        idx_ref[0] = jnp.min(cand)

    out_ref[...] = _view1d(jnp.zeros((A, 128), jnp.int32))

    @pl.when((i == NB - 1) & (idx_ref[0] >= (NB - 1) * CHB))
    def _tail_onehot():
        out_ref[...] = _view1d((_lin(i) == idx_ref[0]).astype(jnp.int32))


def _patch_body(z_ref, idx_ref, out_ref, buf, sem):
    idx = idx_ref[0]
    base = pl.multiple_of((idx // PATCH) * PATCH, PATCH)
    off = idx - base
    pos = lax.broadcasted_iota(jnp.int32, (PATCH,), 0)
    buf[...] = (pos == off).astype(jnp.int32)

    @pl.when(idx < (NB - 1) * CHB)
    def _dma():
        cp = pltpu.make_async_copy(buf, out_ref.at[pl.ds(base, PATCH)], sem)
        cp.start()
        cp.wait()


def kernel(x):
    zeros, idx = pl.pallas_call(
        _k1_body,
        grid=(NB,),
        in_specs=[pl.BlockSpec((CHB,), lambda i: (i,))],
        out_specs=[
            pl.BlockSpec((CHB,), lambda i: (i,)),
            pl.BlockSpec(memory_space=pltpu.SMEM),
        ],
        out_shape=[
            jax.ShapeDtypeStruct((N,), jnp.int32),
            jax.ShapeDtypeStruct((1,), jnp.int32),
        ],
        scratch_shapes=[
            pltpu.SMEM((1,), jnp.float32),
            pltpu.SMEM((1,), jnp.float32),
        ],
    )(x)
    out = pl.pallas_call(
        _patch_body,
        in_specs=[
            pl.BlockSpec(memory_space=pl.ANY),
            pl.BlockSpec(memory_space=pltpu.SMEM),
        ],
        out_specs=pl.BlockSpec(memory_space=pl.ANY),
        out_shape=jax.ShapeDtypeStruct((N,), jnp.int32),
        scratch_shapes=[
            pltpu.VMEM((PATCH,), jnp.int32),
            pltpu.SemaphoreType.DMA,
        ],
        input_output_aliases={0: 0},
    )(zeros, idx)
    return out


# speedup vs baseline: 1.3808x; 1.3808x over previous
import jax
import jax.numpy as jnp
from jax import lax
from jax.experimental import pallas as pl
from jax.experimental.pallas import tpu as pltpu

N = 1000000
CHB = 131072
NB = 8
A = CHB // 128
BIG = 2**30
PATCH = 128


def _view2d(v):
    return pltpu.einshape("(ab)->ab", v, a=A, b=128,
                          assert_is_tile_preserving=True)


def _view1d(v):
    return pltpu.einshape("ab->(ab)", v, assert_is_tile_preserving=True)


def _lin(i):
    r = lax.broadcasted_iota(jnp.int32, (A, 128), 0)
    c = lax.broadcasted_iota(jnp.int32, (A, 128), 1)
    return r * 128 + c + i * CHB


def _k1_body(x_ref, out_ref, idx_ref, max_ref, lm_ref):
    i = pl.program_id(0)

    @pl.when(i < NB - 1)
    def _plain_max():
        lm_ref[0] = jnp.max(_view2d(x_ref[...]))

    @pl.when(i == NB - 1)
    def _masked_max():
        xv = _view2d(x_ref[...])
        lm_ref[0] = jnp.max(jnp.where(_lin(i) < N, xv, -jnp.inf))

    lm = lm_ref[0]

    @pl.when((i == 0) | (lm > max_ref[0]))
    def _new_max():
        xv = _view2d(x_ref[...])
        cand = jnp.where(xv == lm, _lin(i), BIG)
        max_ref[0] = lm
        idx_ref[0] = jnp.min(cand)

    out_ref[...] = _view1d(jnp.zeros((A, 128), jnp.int32))

    @pl.when((i == NB - 1) & (idx_ref[0] >= (NB - 1) * CHB))
    def _tail_onehot():
        out_ref[...] = _view1d((_lin(i) == idx_ref[0]).astype(jnp.int32))


def _patch_body(z_ref, idx_ref, out_ref, buf, sem):
    idx = idx_ref[0]
    base = pl.multiple_of((idx // PATCH) * PATCH, PATCH)
    off = idx - base
    pos = lax.broadcasted_iota(jnp.int32, (PATCH,), 0)
    buf[...] = (pos == off).astype(jnp.int32)

    @pl.when(idx < (NB - 1) * CHB)
    def _dma():
        cp = pltpu.make_async_copy(buf, out_ref.at[pl.ds(base, PATCH)], sem)
        cp.start()
        cp.wait()


def kernel(x):
    zeros, idx = pl.pallas_call(
        _k1_body,
        grid=(NB,),
        in_specs=[pl.BlockSpec((CHB,), lambda i: (i,))],
        out_specs=[
            pl.BlockSpec((CHB,), lambda i: (i,)),
            pl.BlockSpec(memory_space=pltpu.SMEM),
        ],
        out_shape=[
            jax.ShapeDtypeStruct((N,), jnp.int32),
            jax.ShapeDtypeStruct((1,), jnp.int32),
        ],
        scratch_shapes=[
            pltpu.SMEM((1,), jnp.float32),
            pltpu.SMEM((1,), jnp.float32),
        ],
    )(x)
    out = pl.pallas_call(
        _patch_body,
        in_specs=[
            pl.BlockSpec(memory_space=pl.ANY),
            pl.BlockSpec(memory_space=pltpu.SMEM),
        ],
        out_specs=pl.BlockSpec(memory_space=pl.ANY),
        out_shape=jax.ShapeDtypeStruct((N,), jnp.int32),
        scratch_shapes=[
            pltpu.VMEM((PATCH,), jnp.int32),
            pltpu.SemaphoreType.DMA,
        ],
        input_output_aliases={0: 0},
    )(zeros, idx)
    return out


# speedup vs baseline: 1.7516x; 1.2686x over previous
import jax
import jax.numpy as jnp
from jax import lax
from jax.experimental import pallas as pl
from jax.experimental.pallas import tpu as pltpu

N = 1000000
CHB = 262144
NB = 4
A = CHB // 128
BIG = 2**30
PATCH = 128


def _view2d(v):
    return pltpu.einshape("(ab)->ab", v, a=A, b=128,
                          assert_is_tile_preserving=True)


def _view1d(v):
    return pltpu.einshape("ab->(ab)", v, assert_is_tile_preserving=True)


def _lin(i):
    r = lax.broadcasted_iota(jnp.int32, (A, 128), 0)
    c = lax.broadcasted_iota(jnp.int32, (A, 128), 1)
    return r * 128 + c + i * CHB


def _k1_body(x_ref, out_ref, idx_ref, max_ref, lm_ref):
    i = pl.program_id(0)

    @pl.when(i < NB - 1)
    def _plain_max():
        lm_ref[0] = jnp.max(_view2d(x_ref[...]))

    @pl.when(i == NB - 1)
    def _masked_max():
        xv = _view2d(x_ref[...])
        lm_ref[0] = jnp.max(jnp.where(_lin(i) < N, xv, -jnp.inf))

    lm = lm_ref[0]

    @pl.when((i == 0) | (lm > max_ref[0]))
    def _new_max():
        xv = _view2d(x_ref[...])
        cand = jnp.where(xv == lm, _lin(i), BIG)
        max_ref[0] = lm
        idx_ref[0] = jnp.min(cand)

    out_ref[...] = _view1d(jnp.zeros((A, 128), jnp.int32))

    @pl.when((i == NB - 1) & (idx_ref[0] >= (NB - 1) * CHB))
    def _tail_onehot():
        out_ref[...] = _view1d((_lin(i) == idx_ref[0]).astype(jnp.int32))


def _patch_body(z_ref, idx_ref, out_ref, buf, sem):
    idx = idx_ref[0]
    base = pl.multiple_of((idx // PATCH) * PATCH, PATCH)
    off = idx - base
    pos = lax.broadcasted_iota(jnp.int32, (PATCH,), 0)
    buf[...] = (pos == off).astype(jnp.int32)

    @pl.when(idx < (NB - 1) * CHB)
    def _dma():
        cp = pltpu.make_async_copy(buf, out_ref.at[pl.ds(base, PATCH)], sem)
        cp.start()
        cp.wait()


def kernel(x):
    zeros, idx = pl.pallas_call(
        _k1_body,
        grid=(NB,),
        in_specs=[pl.BlockSpec((CHB,), lambda i: (i,))],
        out_specs=[
            pl.BlockSpec((CHB,), lambda i: (i,)),
            pl.BlockSpec(memory_space=pltpu.SMEM),
        ],
        out_shape=[
            jax.ShapeDtypeStruct((N,), jnp.int32),
            jax.ShapeDtypeStruct((1,), jnp.int32),
        ],
        scratch_shapes=[
            pltpu.SMEM((1,), jnp.float32),
            pltpu.SMEM((1,), jnp.float32),
        ],
    )(x)
    out = pl.pallas_call(
        _patch_body,
        in_specs=[
            pl.BlockSpec(memory_space=pl.ANY),
            pl.BlockSpec(memory_space=pltpu.SMEM),
        ],
        out_specs=pl.BlockSpec(memory_space=pl.ANY),
        out_shape=jax.ShapeDtypeStruct((N,), jnp.int32),
        scratch_shapes=[
            pltpu.VMEM((PATCH,), jnp.int32),
            pltpu.SemaphoreType.DMA,
        ],
        input_output_aliases={0: 0},
    )(zeros, idx)
    return out


# speedup vs baseline: 1.9394x; 1.1072x over previous
import jax
import jax.numpy as jnp
from jax import lax
from jax.experimental import pallas as pl
from jax.experimental.pallas import tpu as pltpu

N = 1000000
CHB = 524288
NB = 2
A = CHB // 128
BIG = 2**30
PATCH = 128


def _view2d(v):
    return pltpu.einshape("(ab)->ab", v, a=A, b=128,
                          assert_is_tile_preserving=True)


def _view1d(v):
    return pltpu.einshape("ab->(ab)", v, assert_is_tile_preserving=True)


def _lin(i):
    r = lax.broadcasted_iota(jnp.int32, (A, 128), 0)
    c = lax.broadcasted_iota(jnp.int32, (A, 128), 1)
    return r * 128 + c + i * CHB


def _k1_body(x_ref, out_ref, idx_ref, max_ref, lm_ref):
    i = pl.program_id(0)

    @pl.when(i < NB - 1)
    def _plain_max():
        lm_ref[0] = jnp.max(_view2d(x_ref[...]))

    @pl.when(i == NB - 1)
    def _masked_max():
        xv = _view2d(x_ref[...])
        lm_ref[0] = jnp.max(jnp.where(_lin(i) < N, xv, -jnp.inf))

    lm = lm_ref[0]

    @pl.when((i == 0) | (lm > max_ref[0]))
    def _new_max():
        xv = _view2d(x_ref[...])
        cand = jnp.where(xv == lm, _lin(i), BIG)
        max_ref[0] = lm
        idx_ref[0] = jnp.min(cand)

    out_ref[...] = _view1d(jnp.zeros((A, 128), jnp.int32))

    @pl.when((i == NB - 1) & (idx_ref[0] >= (NB - 1) * CHB))
    def _tail_onehot():
        out_ref[...] = _view1d((_lin(i) == idx_ref[0]).astype(jnp.int32))


def _patch_body(z_ref, idx_ref, out_ref, buf, sem):
    idx = idx_ref[0]
    base = pl.multiple_of((idx // PATCH) * PATCH, PATCH)
    off = idx - base
    pos = lax.broadcasted_iota(jnp.int32, (PATCH,), 0)
    buf[...] = (pos == off).astype(jnp.int32)

    @pl.when(idx < (NB - 1) * CHB)
    def _dma():
        cp = pltpu.make_async_copy(buf, out_ref.at[pl.ds(base, PATCH)], sem)
        cp.start()
        cp.wait()


def kernel(x):
    zeros, idx = pl.pallas_call(
        _k1_body,
        grid=(NB,),
        in_specs=[pl.BlockSpec((CHB,), lambda i: (i,))],
        out_specs=[
            pl.BlockSpec((CHB,), lambda i: (i,)),
            pl.BlockSpec(memory_space=pltpu.SMEM),
        ],
        out_shape=[
            jax.ShapeDtypeStruct((N,), jnp.int32),
            jax.ShapeDtypeStruct((1,), jnp.int32),
        ],
        scratch_shapes=[
            pltpu.SMEM((1,), jnp.float32),
            pltpu.SMEM((1,), jnp.float32),
        ],
    )(x)
    out = pl.pallas_call(
        _patch_body,
        in_specs=[
            pl.BlockSpec(memory_space=pl.ANY),
            pl.BlockSpec(memory_space=pltpu.SMEM),
        ],
        out_specs=pl.BlockSpec(memory_space=pl.ANY),
        out_shape=jax.ShapeDtypeStruct((N,), jnp.int32),
        scratch_shapes=[
            pltpu.VMEM((PATCH,), jnp.int32),
            pltpu.SemaphoreType.DMA,
        ],
        input_output_aliases={0: 0},
    )(zeros, idx)
    return out


# speedup vs baseline: 2.0309x; 1.0472x over previous
import jax
import jax.numpy as jnp
from jax import lax
from jax.experimental import pallas as pl
from jax.experimental.pallas import tpu as pltpu

N = 1000000
CHB = 524288
NB = 2
A = CHB // 128
BIG = 2**30


def _view2d(v):
    return pltpu.einshape("(ab)->ab", v, a=A, b=128,
                          assert_is_tile_preserving=True)


def _view1d(v):
    return pltpu.einshape("ab->(ab)", v, assert_is_tile_preserving=True)


def _lin(i):
    r = lax.broadcasted_iota(jnp.int32, (A, 128), 0)
    c = lax.broadcasted_iota(jnp.int32, (A, 128), 1)
    return r * 128 + c + i * CHB


def _body(x_ref, out_ref, idx_ref, max_ref, lm_ref):
    i = pl.program_id(0)

    @pl.when(i < NB - 1)
    def _plain_max():
        lm_ref[0] = jnp.max(_view2d(x_ref[...]))

    @pl.when(i == NB - 1)
    def _masked_max():
        xv = _view2d(x_ref[...])
        lm_ref[0] = jnp.max(jnp.where(_lin(i) < N, xv, -jnp.inf))

    @pl.when(i < NB)
    def _phase1():
        lm = lm_ref[0]

        @pl.when((i == 0) | (lm > max_ref[0]))
        def _new_max():
            xv = _view2d(x_ref[...])
            cand = jnp.where(xv == lm, _lin(i), BIG)
            max_ref[0] = lm
            idx_ref[0] = jnp.min(cand)

    @pl.when(i >= NB)
    def _phase2():
        j = i - NB
        out_ref[...] = _view1d((_lin(j) == idx_ref[0]).astype(jnp.int32))


def kernel(x):
    return pl.pallas_call(
        _body,
        grid=(2 * NB,),
        in_specs=[pl.BlockSpec((CHB,), lambda i: (jnp.minimum(i, NB - 1),))],
        out_specs=pl.BlockSpec((CHB,), lambda i: (jnp.maximum(i - NB, 0),)),
        out_shape=jax.ShapeDtypeStruct((N,), jnp.int32),
        scratch_shapes=[
            pltpu.SMEM((1,), jnp.int32),
            pltpu.SMEM((1,), jnp.float32),
            pltpu.SMEM((1,), jnp.float32),
        ],
    )(x)
